# baseline (speedup 1.0000x reference)
import functools

import jax
import jax.numpy as jnp
import numpy as np
from jax import lax
from jax.experimental import pallas as pl
from jax.experimental.pallas import tpu as pltpu
from jax.experimental.pallas import tpu_sc as plsc

B, T, D = 4, 8192, 768
KEEP = 4096
ROWS = B * KEEP

_info = plsc.get_sparse_core_info()
_NC = _info.num_cores
_NW = _NC * _info.num_subcores
ROWS_PER_W = ROWS // _NW
CHUNK = 16
NCHUNK = ROWS_PER_W // CHUNK
NBUF = 8


@functools.partial(
    pl.kernel,
    mesh=plsc.VectorSubcoreMesh(core_axis_name="c", subcore_axis_name="s"),
    out_type=jax.ShapeDtypeStruct((ROWS, D), jnp.float32),
    scratch_types=(
        [pltpu.VMEM((ROWS_PER_W,), jnp.int32)]
        + [pltpu.VMEM((CHUNK, D), jnp.float32)] * NBUF
        + [pltpu.SemaphoreType.DMA] * (2 * NBUF)
    ),
)
def _gather_rows(table_hbm, idx_hbm, out_hbm, idx_v, *scr):
    bufs = scr[:NBUF]
    gsems = scr[NBUF:2 * NBUF]
    ssems = scr[2 * NBUF:]
    wid = lax.axis_index("s") * _NC + lax.axis_index("c")
    base = wid * ROWS_PER_W
    pltpu.sync_copy(idx_hbm.at[pl.ds(base, ROWS_PER_W)], idx_v)
    gathers = [None] * NBUF
    stores = [None] * NBUF
    for c in range(NCHUNK + NBUF - 1):
        if c < NCHUNK:
            s = c % NBUF
            if stores[s] is not None:
                stores[s].wait()
            gathers[s] = pltpu.async_copy(
                table_hbm.at[idx_v.at[pl.ds(c * CHUNK, CHUNK)]],
                bufs[s], gsems[s],
            )
        d = c - (NBUF - 1)
        if d >= 0:
            s = d % NBUF
            gathers[s].wait()
            stores[s] = pltpu.async_copy(
                bufs[s], out_hbm.at[pl.ds(base + d * CHUNK, CHUNK)], ssems[s]
            )
    for s in range(NBUF):
        if stores[s] is not None:
            stores[s].wait()


def _compute_gidx():
    rand = jax.random.normal(jax.random.key(42), (B, T - 1), dtype=jnp.float32)
    _, keep = jax.lax.top_k(rand, KEEP - 1)
    fidx = jnp.concatenate(
        [jnp.zeros((B, 1), jnp.int32), keep.astype(jnp.int32) + 1], axis=1
    )
    return (fidx + (jnp.arange(B, dtype=jnp.int32) * T)[:, None]).reshape(ROWS)


_GIDX = np.asarray(jax.jit(_compute_gidx)())


def kernel(x):
    out = _gather_rows(x.reshape(B * T, D), jnp.asarray(_GIDX))
    return out.reshape(B, KEEP, D)


# speedup vs baseline: 1.0006x; 1.0006x over previous
import functools

import jax
import jax.numpy as jnp
import numpy as np
from jax import lax
from jax.experimental import pallas as pl
from jax.experimental.pallas import tpu as pltpu
from jax.experimental.pallas import tpu_sc as plsc

B, T, D = 4, 8192, 768
KEEP = 4096
ROWS = B * KEEP

_info = plsc.get_sparse_core_info()
_NC = _info.num_cores
_NW = _NC * _info.num_subcores
ROWS_PER_W = ROWS // _NW
CHUNK = 32
NCHUNK = ROWS_PER_W // CHUNK
NBUF = 5


@functools.partial(
    pl.kernel,
    mesh=plsc.VectorSubcoreMesh(core_axis_name="c", subcore_axis_name="s"),
    out_type=jax.ShapeDtypeStruct((ROWS, D), jnp.float32),
    scratch_types=(
        [pltpu.VMEM((ROWS_PER_W,), jnp.int32)]
        + [pltpu.VMEM((CHUNK, D), jnp.float32)] * NBUF
        + [pltpu.SemaphoreType.DMA] * (2 * NBUF)
    ),
)
def _gather_rows(table_hbm, idx_hbm, out_hbm, idx_v, *scr):
    bufs = scr[:NBUF]
    gsems = scr[NBUF:2 * NBUF]
    ssems = scr[2 * NBUF:]
    wid = lax.axis_index("s") * _NC + lax.axis_index("c")
    base = wid * ROWS_PER_W
    pltpu.sync_copy(idx_hbm.at[pl.ds(base, ROWS_PER_W)], idx_v)
    gathers = [None] * NBUF
    stores = [None] * NBUF
    for c in range(NCHUNK + NBUF - 1):
        if c < NCHUNK:
            s = c % NBUF
            if stores[s] is not None:
                stores[s].wait()
            gathers[s] = pltpu.async_copy(
                table_hbm.at[idx_v.at[pl.ds(c * CHUNK, CHUNK)]],
                bufs[s], gsems[s],
            )
        d = c - (NBUF - 1)
        if d >= 0:
            s = d % NBUF
            gathers[s].wait()
            stores[s] = pltpu.async_copy(
                bufs[s], out_hbm.at[pl.ds(base + d * CHUNK, CHUNK)], ssems[s]
            )
    for s in range(NBUF):
        if stores[s] is not None:
            stores[s].wait()


def _compute_gidx():
    rand = jax.random.normal(jax.random.key(42), (B, T - 1), dtype=jnp.float32)
    _, keep = jax.lax.top_k(rand, KEEP - 1)
    fidx = jnp.concatenate(
        [jnp.zeros((B, 1), jnp.int32), keep.astype(jnp.int32) + 1], axis=1
    )
    return (fidx + (jnp.arange(B, dtype=jnp.int32) * T)[:, None]).reshape(ROWS)


_GIDX = np.asarray(jax.jit(_compute_gidx)())


def kernel(x):
    out = _gather_rows(x.reshape(B * T, D), jnp.asarray(_GIDX))
    return out.reshape(B, KEEP, D)
